# Initial kernel scaffold; baseline (speedup 1.0000x reference)
import jax
import jax.numpy as jnp
from jax.experimental import pallas as pl


def kernel(x):
    raise NotImplementedError("write your pallas kernel here")


# speedup vs baseline: 3.1103x; 3.1103x over previous
import numpy as np

import jax
import jax.numpy as jnp
from jax import lax
from jax.experimental import pallas as pl
from jax.experimental.pallas import tpu as pltpu
from jax.experimental.pallas import tpu_sc as plsc

ROWS = 128
N = 32768
L = 16
NV = N // L
NC = 2
NS = 16
NW = NC * NS
RPW = ROWS // NW

NB = 2048
SHIFTS = (0, 11, 22)
MASKS = (2047, 2047, 1023)
NBINS = (2048, 2048, 1024)

MININT = np.int32(-2147483648)


def _to_key(v):
    m = v >> 31
    return v ^ (m | MININT)


def _from_key(k):
    m = k >> 31
    return k ^ (~m | MININT)


def _digit(k, p):
    return lax.shift_right_logical(k, jnp.int32(SHIFTS[p])) & jnp.int32(MASKS[p])


def _zero_hist(hist, nbins):
    zeros = jnp.zeros((L,), jnp.int32)

    def body(i, c):
        hist[pl.ds(i * L, L)] = zeros
        return c

    lax.fori_loop(0, nbins // L, body, 0)


def _exclusive_scan(hist, nbins):
    def body(i, carry):
        h = hist[pl.ds(i * L, L)]
        inc = plsc.cumsum(h)
        hist[pl.ds(i * L, L)] = inc - h + carry
        return carry + jnp.sum(h)

    lax.fori_loop(0, nbins // L, body, jnp.int32(0))


def _body(x_hbm, out_hbm, buf_a, buf_b, hist_a, hist_b):
    wid = lax.axis_index("s") * NC + lax.axis_index("c")

    def row_body(r, c0):
        row = wid * RPW + r

        pltpu.sync_copy(x_hbm.at[row], buf_a)
        _zero_hist(hist_a, NBINS[0])

        def sweep0(i, c):
            v = buf_a[pl.ds(i * L, L)]
            k = _to_key(v)
            buf_a[pl.ds(i * L, L)] = k
            d = _digit(k, 0)
            cnt, lastm = plsc.scan_count(d)
            plsc.addupdate_scatter(hist_a, [d], cnt, mask=lastm)
            return c

        lax.fori_loop(0, NV, sweep0, 0)

        def permute(p, src, dst, hist, hist_next):
            _exclusive_scan(hist, NBINS[p])
            if hist_next is not None:
                _zero_hist(hist_next, NBINS[p + 1])

            def sweep(i, c):
                k = src[pl.ds(i * L, L)]
                d = _digit(k, p)
                cnt, lastm = plsc.scan_count(d)
                base = plsc.load_gather(hist, [d])
                off = base + cnt - 1
                val = k if p < 2 else _from_key(k)
                plsc.store_scatter(dst, [off], val)
                plsc.store_scatter(hist, [d], base + cnt, mask=lastm)
                if hist_next is not None:
                    d2 = _digit(k, p + 1)
                    cnt2, lastm2 = plsc.scan_count(d2)
                    plsc.addupdate_scatter(hist_next, [d2], cnt2, mask=lastm2)
                return c

            lax.fori_loop(0, NV, sweep, 0)

        permute(0, buf_a, buf_b, hist_a, hist_b)
        permute(1, buf_b, buf_a, hist_b, hist_a)
        permute(2, buf_a, buf_b, hist_a, None)

        pltpu.sync_copy(buf_b, out_hbm.at[row])
        return c0

    lax.fori_loop(0, RPW, row_body, 0)


@jax.jit
def kernel(x):
    xi = lax.bitcast_convert_type(x, jnp.int32)
    mesh = plsc.VectorSubcoreMesh(core_axis_name="c", subcore_axis_name="s")
    sort_rows = pl.kernel(
        _body,
        out_type=jax.ShapeDtypeStruct((ROWS, N), jnp.int32),
        mesh=mesh,
        compiler_params=pltpu.CompilerParams(needs_layout_passes=False),
        scratch_types=[
            pltpu.VMEM((N,), jnp.int32),
            pltpu.VMEM((N,), jnp.int32),
            pltpu.VMEM((NB,), jnp.int32),
            pltpu.VMEM((NB,), jnp.int32),
        ],
    )
    oi = sort_rows(xi)
    return lax.bitcast_convert_type(oi, jnp.float32)
